# baseline (speedup 1.0000x reference)
import functools

import jax
import jax.numpy as jnp
from jax import lax
from jax.experimental import pallas as pl
from jax.experimental.pallas import tpu as pltpu
from jax.experimental.pallas import tpu_sc as plsc

_W = 128


def kernel(x, emb):
    B, H = x.shape
    V, D = emb.shape
    n = B * H
    info = plsc.get_sparse_core_info()
    nw = info.num_cores * info.num_subcores
    nbt = B // _W
    nd8 = D // 8
    assert nbt == nw and D % 8 == 0 and H % 2 == 0

    xt = x.T.reshape(H, nbt, _W).astype(jnp.int32)

    mesh = plsc.VectorSubcoreMesh(core_axis_name="core", subcore_axis_name="subcore")

    @functools.partial(
        pl.kernel,
        out_type=jax.ShapeDtypeStruct((H, nd8, nbt, 8, _W), emb.dtype),
        mesh=mesh,
        scratch_types=[
            pltpu.VMEM((H, _W), jnp.int32),
            pltpu.VMEM((2, _W, D), jnp.float32),
            pltpu.VMEM((2, nd8, 8, _W), jnp.float32),
            pltpu.SemaphoreType.DMA((2,)),
            pltpu.SemaphoreType.DMA((2,)),
        ],
        compiler_params=pltpu.CompilerParams(
            use_tc_tiling_on_sc=False, needs_layout_passes=False),
    )
    def gather_kernel(emb_hbm, idx_hbm, out_hbm, idx_v, g_v, t_v, gsem, ssem):
        wid = lax.axis_index("subcore") * info.num_cores + lax.axis_index("core")
        pltpu.sync_copy(idx_hbm.at[:, wid], idx_v)

        def gather(h, b):
            return pltpu.make_async_copy(
                emb_hbm.at[idx_v.at[h]], g_v.at[b], gsem.at[b])

        def scatter(h, b):
            return pltpu.make_async_copy(
                t_v.at[b], out_hbm.at[h, :, wid], ssem.at[b])

        def transpose(b):
            @pl.loop(0, nd8)
            def _(dt):
                for b8 in range(_W // 16):
                    rows = b8 * 16 + lax.iota(jnp.int32, 16)
                    for d8 in range(8):
                        col = jnp.full((16,), dt * 8 + d8, jnp.int32)
                        vec = plsc.load_gather(g_v.at[b], [rows, col])
                        t_v[b, dt, d8, pl.ds(b8 * 16, 16)] = vec

        gather(0, 0).start()
        gather(1, 1).start()

        for b in range(2):
            gather(b, b).wait()
            transpose(b)
            scatter(b, b).start()
            gather(2 + b, b).start()

        @pl.loop(2, H - 2, step=2)
        def _(h0):
            for b in range(2):
                h = h0 + b
                gather(h, b).wait()
                scatter(h - 2, b).wait()
                transpose(b)
                scatter(h, b).start()
                gather(h + 2, b).start()

        h0 = H - 2
        for b in range(2):
            h = h0 + b
            gather(h, b).wait()
            scatter(h - 2, b).wait()
            transpose(b)
            scatter(h, b).start()
        for b in range(2):
            scatter(h0 + b, b).wait()

    out5 = gather_kernel(emb, xt)
    return out5.transpose(2, 4, 0, 1, 3).reshape(B, H, D)


# speedup vs baseline: 1.8907x; 1.8907x over previous
import functools

import jax
import jax.numpy as jnp
from jax import lax
from jax.experimental import pallas as pl
from jax.experimental.pallas import tpu as pltpu
from jax.experimental.pallas import tpu_sc as plsc

_W = 128


def kernel(x, emb):
    B, H = x.shape
    V, D = emb.shape
    n = B * H
    info = plsc.get_sparse_core_info()
    nw = info.num_cores * info.num_subcores
    nbt = B // _W
    nd8 = D // 8
    assert nbt == nw and D % 8 == 0 and H % 2 == 0

    xt = x.T.reshape(H, nbt, _W).astype(jnp.int32)

    mesh = plsc.VectorSubcoreMesh(core_axis_name="core", subcore_axis_name="subcore")

    @functools.partial(
        pl.kernel,
        out_type=jax.ShapeDtypeStruct((H, nd8, nbt, 8, _W), emb.dtype),
        mesh=mesh,
        scratch_types=[
            pltpu.VMEM((H, _W), jnp.int32),
            pltpu.VMEM((2, _W, D), jnp.float32),
            pltpu.VMEM((2, nd8, 8, _W), jnp.float32),
            pltpu.SemaphoreType.DMA((2,)),
            pltpu.SemaphoreType.DMA((2,)),
        ],
        compiler_params=pltpu.CompilerParams(
            use_tc_tiling_on_sc=False, needs_layout_passes=False),
    )
    def gather_kernel(emb_hbm, idx_hbm, out_hbm, idx_v, g_v, t_v, gsem, ssem):
        wid = lax.axis_index("subcore") * info.num_cores + lax.axis_index("core")
        pltpu.sync_copy(idx_hbm.at[:, wid], idx_v)

        def gather(h, b):
            return pltpu.make_async_copy(
                emb_hbm.at[idx_v.at[h]], g_v.at[b], gsem.at[b])

        def scatter(h, b):
            return pltpu.make_async_copy(
                t_v.at[b], out_hbm.at[h, :, wid], ssem.at[b])

        lanes = lax.iota(jnp.int32, 16)
        bks = [(lanes + k) & 15 for k in range(16)]
        dvs = [d0 + lanes for d0 in range(0, D, 16)]
        dtvs = [dv >> 3 for dv in dvs]
        d8vs = [dv & 7 for dv in dvs]

        def transpose(b):
            @pl.loop(0, _W, step=16)
            def _(b0):
                bvs = [b0 + bk for bk in bks]
                for di in range(D // 16):
                    for k in range(16):
                        vec = plsc.load_gather(g_v.at[b], [bvs[k], dvs[di]])
                        plsc.store_scatter(
                            t_v.at[b], [dtvs[di], d8vs[di], bvs[k]], vec)

        gather(0, 0).start()
        gather(1, 1).start()

        for b in range(2):
            gather(b, b).wait()
            transpose(b)
            scatter(b, b).start()
            gather(2 + b, b).start()

        @pl.loop(2, H - 2, step=2)
        def _(h0):
            for b in range(2):
                h = h0 + b
                gather(h, b).wait()
                scatter(h - 2, b).wait()
                transpose(b)
                scatter(h, b).start()
                gather(h + 2, b).start()

        h0 = H - 2
        for b in range(2):
            h = h0 + b
            gather(h, b).wait()
            scatter(h - 2, b).wait()
            transpose(b)
            scatter(h, b).start()
        for b in range(2):
            scatter(h0 + b, b).wait()

    out5 = gather_kernel(emb, xt)
    return out5.transpose(2, 4, 0, 1, 3).reshape(B, H, D)
